# Initial kernel scaffold; baseline (speedup 1.0000x reference)
import jax
import jax.numpy as jnp
from jax.experimental import pallas as pl


def kernel(tensor_input, keys, values):
    raise NotImplementedError("write your pallas kernel here")


# speedup vs baseline: 3.7666x; 3.7666x over previous
import functools

import jax
import jax.numpy as jnp
from jax import lax
from jax.experimental import pallas as pl
from jax.experimental.pallas import tpu as pltpu
from jax.experimental.pallas import tpu_sc as plsc

_NC = 2
_NS = 16
_L = 16
_NW = _NC * _NS

_N = 16384
_C = 16
_RPW = _N // _NW
_BLOCKS = _RPW // _L


def _body(x_hbm, keys_hbm, values_hbm, out_hbm, kv_v, vv_v, t_v, x_v, o_v):
    cid = lax.axis_index("c")
    sid = lax.axis_index("s")
    wid = sid * _NC + cid
    base = wid * _RPW

    pltpu.sync_copy(keys_hbm, kv_v)
    pltpu.sync_copy(values_hbm, vv_v)
    pltpu.sync_copy(x_hbm.at[pl.ds(base * _C, _RPW * _C)], x_v)

    lane = lax.iota(jnp.int32, _L)

    kvec = kv_v[...]
    pos = jnp.where(kvec[0] < lane, 1, 0).astype(jnp.int32)
    for k in range(1, _C):
        pos = pos + jnp.where(kvec[k] < lane, 1, 0).astype(jnp.int32)
    pos_c = jnp.minimum(pos, _C - 1)
    key_at = plsc.load_gather(kv_v, [pos_c])
    val_at = plsc.load_gather(vv_v, [pos_c])
    t_v[...] = jnp.where(key_at == lane, val_at, jnp.float32(-1.0))

    col_idx = [jnp.bitwise_and(lane + j, _C - 1) for j in range(_C)]

    def blk(b, carry):
        addr_base = b * (_L * _C) + lane * _C
        bv = plsc.load_gather(x_v, [addr_base + col_idx[0]])
        bi = col_idx[0]
        for j in range(1, _C):
            cj = plsc.load_gather(x_v, [addr_base + col_idx[j]])
            ci = col_idx[j]
            better = (cj > bv) | ((cj == bv) & (ci < bi))
            bv = jnp.where(better, cj, bv)
            bi = jnp.where(better, ci, bi)
        res = plsc.load_gather(t_v, [bi])
        o_v[pl.ds(b * _L, _L)] = res
        return carry

    lax.fori_loop(0, _BLOCKS, blk, 0)

    pltpu.sync_copy(o_v, out_hbm.at[pl.ds(base, _RPW)])


@jax.jit
def _run(x_flat, keys_i32, values):
    return pl.kernel(
        _body,
        out_type=jax.ShapeDtypeStruct((_N,), jnp.float32),
        mesh=plsc.VectorSubcoreMesh(core_axis_name="c", subcore_axis_name="s"),
        compiler_params=pltpu.CompilerParams(needs_layout_passes=False),
        scratch_types=[
            pltpu.VMEM((_C,), jnp.int32),
            pltpu.VMEM((_C,), jnp.float32),
            pltpu.VMEM((_C,), jnp.float32),
            pltpu.VMEM((_RPW * _C,), jnp.float32),
            pltpu.VMEM((_RPW,), jnp.float32),
        ],
    )(x_flat, keys_i32, values)


def kernel(tensor_input, keys, values):
    x_flat = jnp.reshape(tensor_input, (-1,))
    return _run(x_flat, keys.astype(jnp.int32), values)


# speedup vs baseline: 3.8414x; 1.0199x over previous
import jax
import jax.numpy as jnp
from jax import lax
from jax.experimental import pallas as pl
from jax.experimental.pallas import tpu as pltpu
from jax.experimental.pallas import tpu_sc as plsc

_NC = 2
_NS = 16
_L = 16
_NW = _NC * _NS

_N = 16384
_C = 16
_RPW = _N // _NW
_BLOCKS = _RPW // _L
_BIG = 1 << 20


def _body(x_hbm, keys_hbm, values_hbm, out_hbm, kv_v, vv_v, t_v, x_v, o_v):
    cid = lax.axis_index("c")
    sid = lax.axis_index("s")
    wid = sid * _NC + cid
    base = wid * _RPW

    pltpu.sync_copy(keys_hbm, kv_v)
    pltpu.sync_copy(values_hbm, vv_v)
    pltpu.sync_copy(x_hbm.at[pl.ds(base * _C, _RPW * _C)], x_v)

    lane = lax.iota(jnp.int32, _L)

    kvec = kv_v[...]
    pos = jnp.where(kvec[0] < lane, 1, 0).astype(jnp.int32)
    for k in range(1, _C):
        pos = pos + jnp.where(kvec[k] < lane, 1, 0).astype(jnp.int32)
    pos_c = jnp.minimum(pos, _C - 1)
    key_at = plsc.load_gather(kv_v, [pos_c])
    val_at = plsc.load_gather(vv_v, [pos_c])
    t_v[...] = jnp.where(key_at == lane, val_at, jnp.float32(-1.0))

    cols = [jnp.bitwise_and(lane + j, _C - 1) for j in range(_C)]
    row0 = lane * _C

    @plsc.parallel_loop(0, _BLOCKS, unroll=2)
    def _blk(b):
        addr0 = b * (_L * _C) + row0
        vs = [plsc.load_gather(x_v, [addr0 + cols[j]]) for j in range(_C)]
        m = vs
        while len(m) > 1:
            m = [jnp.maximum(m[i], m[i + 1]) for i in range(0, len(m), 2)]
        mx = m[0]
        bi = jnp.where(vs[0] == mx, cols[0], _BIG)
        for j in range(1, _C):
            bi = jnp.minimum(bi, jnp.where(vs[j] == mx, cols[j], _BIG))
        res = plsc.load_gather(t_v, [bi])
        o_v[pl.ds(b * _L, _L)] = res

    pltpu.sync_copy(o_v, out_hbm.at[pl.ds(base, _RPW)])


@jax.jit
def _run(x_flat, keys_i32, values):
    return pl.kernel(
        _body,
        out_type=jax.ShapeDtypeStruct((_N,), jnp.float32),
        mesh=plsc.VectorSubcoreMesh(core_axis_name="c", subcore_axis_name="s"),
        compiler_params=pltpu.CompilerParams(needs_layout_passes=False),
        scratch_types=[
            pltpu.VMEM((_C,), jnp.int32),
            pltpu.VMEM((_C,), jnp.float32),
            pltpu.VMEM((_C,), jnp.float32),
            pltpu.VMEM((_RPW * _C,), jnp.float32),
            pltpu.VMEM((_RPW,), jnp.float32),
        ],
    )(x_flat, keys_i32, values)


def kernel(tensor_input, keys, values):
    x_flat = jnp.reshape(tensor_input, (-1,))
    return _run(x_flat, keys.astype(jnp.int32), values)
